# baseline (speedup 1.0000x reference)
import functools

import jax
import jax.numpy as jnp
from jax import lax
from jax.experimental import pallas as pl
from jax.experimental.pallas import tpu as pltpu
from jax.experimental.pallas import tpu_sc as plsc

DIM = 64
DIMP = 128
NUM_EMB = 1000000
ROWS = 16384
COLS = 26
B_TOTAL = ROWS * COLS
NUM_WORKERS = 32
B_PER_W = B_TOTAL // NUM_WORKERS
CHUNK = 128
N_CHUNKS = B_PER_W // CHUNK
GROUPS_PER_ROW = ROWS // CHUNK
NBUF = 4
NST = 2

T_BLOCKS = NUM_EMB // CHUNK
T_PER_W = T_BLOCKS // NUM_WORKERS
T_EXTRA = T_BLOCKS - T_PER_W * NUM_WORKERS
T_TAIL0 = T_BLOCKS * CHUNK

_mesh = plsc.VectorSubcoreMesh(core_axis_name="c", subcore_axis_name="s")


@functools.partial(
    pl.kernel,
    mesh=_mesh,
    out_type=jax.ShapeDtypeStruct((NUM_EMB, DIMP), jnp.float32),
    scratch_types=[
        pltpu.VMEM((2, DIM, CHUNK), jnp.float32),
        pltpu.VMEM((2, CHUNK, DIMP), jnp.float32),
    ]
    + [pltpu.SemaphoreType.DMA] * 2
    + [pltpu.SemaphoreType.DMA] * 2,
    compiler_params=pltpu.CompilerParams(
        use_tc_tiling_on_sc=True,
        needs_layout_passes=False,
        disable_bounds_checks=True,
    ),
)
def _transpose_kernel(tt_hbm, tail_hbm, out_hbm, in_v, out_v, *sems):
    isems = sems[:2]
    osems = sems[2:]
    wid = lax.axis_index("s") * 2 + lax.axis_index("c")
    n_extra = jnp.where(wid < T_EXTRA, 1, 0)
    start = T_PER_W * wid + jnp.minimum(wid, T_EXTRA)
    iota = lax.iota(jnp.int32, 16)

    def fire_in(j, ib):
        c0 = (start + j) * CHUNK
        pltpu.async_copy(tt_hbm.at[:, pl.ds(c0, CHUNK)], in_v.at[ib], isems[ib])

    def drain_in(ib):
        pltpu.make_async_copy(
            tt_hbm.at[:, pl.ds(0, CHUNK)], in_v.at[ib], isems[ib]
        ).wait()

    def transpose(ib, ob):
        def col_block(i, carry):
            row_idx = i * 16 + iota
            for d in range(DIM):
                vec = in_v[ib, d, pl.ds(i * 16, 16)]
                plsc.store_scatter(
                    out_v.at[ob],
                    [row_idx, jnp.full((16,), d, jnp.int32)],
                    vec,
                )
            return carry

        lax.fori_loop(0, CHUNK // 16, col_block, 0)

    def fire_out(j, ob):
        r0 = (start + j) * CHUNK
        pltpu.async_copy(out_v.at[ob], out_hbm.at[pl.ds(r0, CHUNK)], osems[ob])

    def drain_out(ob):
        pltpu.make_async_copy(
            out_v.at[ob], out_hbm.at[pl.ds(0, CHUNK)], osems[ob]
        ).wait()

    fire_in(0, 0)
    fire_in(1, 1)

    for j in range(2):
        drain_in(j)
        transpose(j, j)
        fire_out(j, j)
        fire_in(j + 2, j)

    def body(o, carry):
        for ib in range(2):
            j = o * 2 + ib
            drain_in(ib)
            drain_out(ib)
            transpose(ib, ib)
            fire_out(j, ib)
            fire_in(j + 2, ib)
        return carry

    lax.fori_loop(1, T_PER_W // 2 - 1, body, 0)

    for jj in range(2):
        j = T_PER_W - 2 + jj
        ib = j % 2
        drain_in(ib)
        drain_out(ib)
        transpose(ib, ib)
        fire_out(j, ib)
    for ob in range(2):
        drain_out(ob)

    @pl.when(n_extra == 1)
    def _():
        pltpu.sync_copy(
            tt_hbm.at[:, pl.ds((start + T_PER_W) * CHUNK, CHUNK)], in_v.at[0]
        )
        transpose(0, 0)
        pltpu.sync_copy(
            out_v.at[0], out_hbm.at[pl.ds((start + T_PER_W) * CHUNK, CHUNK)]
        )

    @pl.when(wid == NUM_WORKERS - 1)
    def _():
        pltpu.sync_copy(tail_hbm, in_v.at[0, pl.ds(0, DIM)])
        pltpu.sync_copy(in_v.at[0, pl.ds(0, DIM)], out_hbm.at[pl.ds(T_TAIL0, DIM)])


@functools.partial(
    pl.kernel,
    mesh=_mesh,
    out_type=jax.ShapeDtypeStruct((COLS, ROWS, DIM), jnp.float32),
    scratch_types=[
        pltpu.VMEM((N_CHUNKS, CHUNK), jnp.int32),
        pltpu.VMEM((NBUF, CHUNK, DIMP), jnp.float32),
        pltpu.VMEM((NST, CHUNK, DIM), jnp.float32),
    ]
    + [pltpu.SemaphoreType.DMA] * NBUF
    + [pltpu.SemaphoreType.DMA] * NST,
    compiler_params=pltpu.CompilerParams(use_tc_tiling_on_sc=True),
)
def _gather_kernel(idx_hbm, table_hbm, out_hbm, idx_v, rows_v, st_v, *sems):
    gsems = sems[:NBUF]
    ssems = sems[NBUF:]
    wid = lax.axis_index("s") * 2 + lax.axis_index("c")
    g_base = wid * N_CHUNKS
    pltpu.sync_copy(idx_hbm.at[wid], idx_v)

    def fire_gather(g, b):
        pltpu.async_copy(table_hbm.at[idx_v.at[g]], rows_v.at[b], gsems[b])

    def drain_gather(b):
        pltpu.make_async_copy(
            table_hbm.at[pl.ds(0, CHUNK)], rows_v.at[b], gsems[b]
        ).wait()

    def compact(b, s):
        def row_block(r0, carry):
            for rr in range(8):
                r = r0 * 8 + rr
                for k in range(DIM // 16):
                    st_v[s, r, pl.ds(k * 16, 16)] = rows_v[b, r, pl.ds(k * 16, 16)]
            return carry

        lax.fori_loop(0, CHUNK // 8, row_block, 0)

    def fire_store(g, s):
        gg = g_base + g
        c = gg // GROUPS_PER_ROW
        b0 = (gg % GROUPS_PER_ROW) * CHUNK
        pltpu.async_copy(
            st_v.at[s],
            out_hbm.at[c, pl.ds(b0, CHUNK)],
            ssems[s],
        )

    def drain_store(s):
        pltpu.make_async_copy(
            st_v.at[s],
            out_hbm.at[0, pl.ds(0, CHUNK)],
            ssems[s],
        ).wait()

    for b in range(NBUF):
        fire_gather(b, b)

    for b in range(NBUF):
        s = b % NST
        drain_gather(b)
        if b >= NST:
            drain_store(s)
        compact(b, s)
        fire_store(b, s)
        fire_gather(b + NBUF, b)

    def outer(o, carry):
        g0 = o * NBUF
        for b in range(NBUF):
            g = g0 + b
            s = b % NST
            drain_gather(b)
            drain_store(s)
            compact(b, s)
            fire_store(g, s)
            fire_gather(g + NBUF, b)
        return carry

    lax.fori_loop(1, N_CHUNKS // NBUF - 1, outer, 0)

    for b in range(NBUF):
        g = N_CHUNKS - NBUF + b
        s = b % NST
        drain_gather(b)
        drain_store(s)
        compact(b, s)
        fire_store(g, s)
    for s in range(NST):
        drain_store(s)


def kernel(x, table):
    idx = x.T.reshape(NUM_WORKERS, N_CHUNKS, CHUNK)
    tail = jnp.pad(table[T_TAIL0:], ((0, 0), (0, DIMP - DIM)))
    tpad = _transpose_kernel(table.T, tail)
    out = _gather_kernel(idx, tpad)
    return out.transpose(1, 0, 2)


# speedup vs baseline: 1.6441x; 1.6441x over previous
import functools

import jax
import jax.numpy as jnp
from jax import lax
from jax.experimental import pallas as pl
from jax.experimental.pallas import tpu as pltpu
from jax.experimental.pallas import tpu_sc as plsc

DIM = 64
DIMP = 128
NUM_EMB = 1000000
ROWS = 16384
COLS = 26
B_TOTAL = ROWS * COLS
NUM_WORKERS = 32
B_PER_W = B_TOTAL // NUM_WORKERS
CHUNK = 128
N_CHUNKS = B_PER_W // CHUNK
GROUPS_PER_ROW = ROWS // CHUNK
NBUF = 4
NST = 2

_mesh = plsc.VectorSubcoreMesh(core_axis_name="c", subcore_axis_name="s")


@functools.partial(
    pl.kernel,
    mesh=_mesh,
    out_type=jax.ShapeDtypeStruct((COLS, ROWS, DIM), jnp.float32),
    scratch_types=[
        pltpu.VMEM((N_CHUNKS, CHUNK), jnp.int32),
        pltpu.VMEM((N_CHUNKS, CHUNK), jnp.int32),
        pltpu.VMEM((NBUF, CHUNK, DIMP), jnp.float32),
        pltpu.VMEM((NST, CHUNK, DIM), jnp.float32),
    ]
    + [pltpu.SemaphoreType.DMA] * NBUF
    + [pltpu.SemaphoreType.DMA] * NST,
    compiler_params=pltpu.CompilerParams(use_tc_tiling_on_sc=True),
)
def _gather_kernel(idx_hbm, table_hbm, out_hbm, idx_v, pidx_v, rows_v, st_v, *sems):
    gsems = sems[:NBUF]
    ssems = sems[NBUF:]
    wid = lax.axis_index("s") * 2 + lax.axis_index("c")
    g_base = wid * N_CHUNKS
    pltpu.sync_copy(idx_hbm.at[wid], idx_v)

    def pair_row(g, carry):
        for k in range(CHUNK // 16):
            sl = pl.ds(k * 16, 16)
            pidx_v[g, sl] = jnp.right_shift(idx_v[g, sl], 1)
        return carry

    lax.fori_loop(0, N_CHUNKS, pair_row, 0)

    def fire_gather(g, b):
        pltpu.async_copy(table_hbm.at[pidx_v.at[g]], rows_v.at[b], gsems[b])

    def drain_gather(b):
        pltpu.make_async_copy(
            table_hbm.at[pl.ds(0, CHUNK)], rows_v.at[b], gsems[b]
        ).wait()

    def compact(g, b, s):
        def row_block(i, carry):
            hv = (idx_v[g, pl.ds(i * 16, 16)] & 1) * DIM
            for rr in range(16):
                r = i * 16 + rr
                off = hv[rr]
                for k in range(DIM // 16):
                    st_v[s, r, pl.ds(k * 16, 16)] = rows_v[
                        b, r, pl.ds(off + k * 16, 16)
                    ]
            return carry

        lax.fori_loop(0, CHUNK // 16, row_block, 0)

    def fire_store(g, s):
        gg = g_base + g
        c = gg // GROUPS_PER_ROW
        b0 = (gg % GROUPS_PER_ROW) * CHUNK
        pltpu.async_copy(
            st_v.at[s],
            out_hbm.at[c, pl.ds(b0, CHUNK)],
            ssems[s],
        )

    def drain_store(s):
        pltpu.make_async_copy(
            st_v.at[s],
            out_hbm.at[0, pl.ds(0, CHUNK)],
            ssems[s],
        ).wait()

    for b in range(NBUF):
        fire_gather(b, b)

    for b in range(NBUF):
        s = b % NST
        drain_gather(b)
        if b >= NST:
            drain_store(s)
        compact(b, b, s)
        fire_store(b, s)
        fire_gather(b + NBUF, b)

    def outer(o, carry):
        g0 = o * NBUF
        for b in range(NBUF):
            g = g0 + b
            s = b % NST
            drain_gather(b)
            drain_store(s)
            compact(g, b, s)
            fire_store(g, s)
            fire_gather(g + NBUF, b)
        return carry

    lax.fori_loop(1, N_CHUNKS // NBUF - 1, outer, 0)

    for b in range(NBUF):
        g = N_CHUNKS - NBUF + b
        s = b % NST
        drain_gather(b)
        drain_store(s)
        compact(g, b, s)
        fire_store(g, s)
    for s in range(NST):
        drain_store(s)


def kernel(x, table):
    idx = x.T.reshape(NUM_WORKERS, N_CHUNKS, CHUNK)
    tpad = table.reshape(NUM_EMB // 2, DIMP)
    out = _gather_kernel(idx, tpad)
    return out.transpose(1, 0, 2)


# speedup vs baseline: 1.8914x; 1.1504x over previous
import functools

import jax
import jax.numpy as jnp
from jax import lax
from jax.experimental import pallas as pl
from jax.experimental.pallas import tpu as pltpu
from jax.experimental.pallas import tpu_sc as plsc

DIM = 64
DIMP = 128
NUM_EMB = 1000000
ROWS = 16384
COLS = 26
B_TOTAL = ROWS * COLS
NUM_WORKERS = 32
B_PER_W = B_TOTAL // NUM_WORKERS
CHUNK = 128
N_CHUNKS = B_PER_W // CHUNK
GROUPS_PER_ROW = ROWS // CHUNK
NBUF = 4
NST = 2

_mesh = plsc.VectorSubcoreMesh(core_axis_name="c", subcore_axis_name="s")


@functools.partial(
    pl.kernel,
    mesh=_mesh,
    out_type=jax.ShapeDtypeStruct((COLS, ROWS, DIM), jnp.float32),
    scratch_types=[
        pltpu.VMEM((N_CHUNKS, CHUNK), jnp.int32),
        pltpu.VMEM((NBUF, CHUNK, DIMP), jnp.float32),
        pltpu.VMEM((NST, CHUNK, DIM), jnp.float32),
    ]
    + [pltpu.SemaphoreType.DMA] * NBUF
    + [pltpu.SemaphoreType.DMA] * NST,
    compiler_params=pltpu.CompilerParams(use_tc_tiling_on_sc=True),
)
def _gather_kernel(idx_hbm, table_hbm, out_hbm, idx_v, rows_v, st_v, *sems):
    gsems = sems[:NBUF]
    ssems = sems[NBUF:]
    wid = lax.axis_index("s") * 2 + lax.axis_index("c")
    g_base = wid * N_CHUNKS
    pltpu.sync_copy(idx_hbm.at[wid], idx_v)

    def fire_gather(g, b):
        pltpu.async_copy(table_hbm.at[idx_v.at[g]], rows_v.at[b], gsems[b])

    def drain_gather(b):
        pltpu.make_async_copy(
            table_hbm.at[pl.ds(0, CHUNK)], rows_v.at[b], gsems[b]
        ).wait()

    def compact(b, s):
        def row_block(r0, carry):
            for rr in range(8):
                r = r0 * 8 + rr
                for k in range(DIM // 16):
                    st_v[s, r, pl.ds(k * 16, 16)] = rows_v[b, r, pl.ds(k * 16, 16)]
            return carry

        lax.fori_loop(0, CHUNK // 8, row_block, 0)

    def fire_store(g, s):
        gg = g_base + g
        c = gg // GROUPS_PER_ROW
        b0 = (gg % GROUPS_PER_ROW) * CHUNK
        pltpu.async_copy(
            st_v.at[s],
            out_hbm.at[c, pl.ds(b0, CHUNK)],
            ssems[s],
        )

    def drain_store(s):
        pltpu.make_async_copy(
            st_v.at[s],
            out_hbm.at[0, pl.ds(0, CHUNK)],
            ssems[s],
        ).wait()

    for b in range(NBUF):
        fire_gather(b, b)

    for b in range(NBUF):
        s = b % NST
        drain_gather(b)
        if b >= NST:
            drain_store(s)
        compact(b, s)
        fire_store(b, s)
        fire_gather(b + NBUF, b)

    def outer(o, carry):
        g0 = o * NBUF
        for b in range(NBUF):
            g = g0 + b
            s = b % NST
            drain_gather(b)
            drain_store(s)
            compact(b, s)
            fire_store(g, s)
            fire_gather(g + NBUF, b)
        return carry

    lax.fori_loop(1, N_CHUNKS // NBUF - 1, outer, 0)

    for b in range(NBUF):
        g = N_CHUNKS - NBUF + b
        s = b % NST
        drain_gather(b)
        drain_store(s)
        compact(b, s)
        fire_store(g, s)
    for s in range(NST):
        drain_store(s)


def kernel(x, table):
    idx = x.T.reshape(NUM_WORKERS, N_CHUNKS, CHUNK)
    tpad = jnp.pad(table, ((0, 0), (0, DIMP - DIM)))
    out = _gather_kernel(idx, tpad)
    return out.transpose(1, 0, 2)
